# baseline (speedup 1.0000x reference)
import functools
import math

import jax
import jax.numpy as jnp
from jax.experimental import pallas as pl
from jax.experimental.pallas import tpu as pltpu

_VOCAB = 64
_S = 16
_H = 32
_NH = 4
_HD = 8
_FF = 64
_NL = 2
_NC = 3
_BB = 512
_G = 4


def _fwd_kernel(tok_ref, te_ref, pe_ref, eg_ref, eb_ref,
                wqkv_ref, bqkv_ref, wo_ref, bo_ref,
                ln1g_ref, ln1b_ref, wf1_ref, bf1_ref, wf2_ref, bf2_ref,
                ln2g_ref, ln2b_ref, wp_ref, bp_ref,
                y_ref, qkv_s, ctx_s, kt_s):
    R = _BB * _S
    scale = 1.0 / math.sqrt(_HD)
    inv_sqrt2 = 1.0 / math.sqrt(2.0)
    GR = _G * _S
    GC = _NH * GR

    f32 = jnp.float32

    def ln(x, g, b):
        inv_h = 1.0 / _H
        mu = jnp.sum(x, axis=-1, keepdims=True) * inv_h
        m2 = jnp.sum(x * x, axis=-1, keepdims=True) * inv_h
        var = jnp.maximum(m2 - mu * mu, 0.0)
        a = jax.lax.rsqrt(var + 1e-12) * g
        return x * a + (b - mu * a)

    r_i = jax.lax.broadcasted_iota(jnp.int32, (GC, _H), 0)
    c_i = jax.lax.broadcasted_iota(jnp.int32, (GC, _H), 1)
    mask_kv = (r_i // GR == c_i // _HD).astype(f32)
    t_r = jax.lax.broadcasted_iota(jnp.int32, (_H, GC), 0)
    t_c = jax.lax.broadcasted_iota(jnp.int32, (_H, GC), 1)
    mask_kt = jnp.where(t_r // _HD == t_c // GR, scale, 0.0).astype(f32)
    sr = jax.lax.broadcasted_iota(jnp.int32, (GR, GC), 0)
    sc = jax.lax.broadcasted_iota(jnp.int32, (GR, GC), 1)
    neg = jnp.where(sr // _S == (sc // _S) % _G, 0.0, -1e30).astype(f32)

    tok = tok_ref[...]
    voc = jax.lax.broadcasted_iota(jnp.int32, (_BB, _S, _VOCAB), 2)
    onehot = (tok[:, :, None] == voc).astype(f32).reshape(R, _VOCAB)
    emb = jnp.dot(onehot, te_ref[...], preferred_element_type=f32)
    pos = jnp.broadcast_to(pe_ref[...][None], (_BB, _S, _H)).reshape(R, _H)
    h = ln(emb + pos, eg_ref[...], eb_ref[...])

    n_grp = _BB // _G
    for l in range(_NL):
        qkv = (jnp.dot(h, wqkv_ref[l], preferred_element_type=f32)
               + bqkv_ref[l])
        qkv_s[...] = qkv
        kt_s[...] = qkv[:, _H:2 * _H].T

        def attn_chunk(c, carry):
            for uu in range(4):
                ck = c * 4 + uu
                ktc = kt_s[:, pl.ds(ck * 2 * GR, 2 * GR)]
                for u in range(2):
                    g = ck * 2 + u
                    base = g * GR
                    blk = qkv_s[pl.ds(base, GR), :]
                    q = blk[:, 0:_H]
                    ktg = ktc[:, u * GR:(u + 1) * GR]
                    kt4 = jnp.concatenate([ktg] * _NH, axis=1) * mask_kt
                    s = jnp.dot(q, kt4, preferred_element_type=f32)
                    p = jnp.exp(s + neg)
                    v4 = jnp.concatenate(
                        [blk[:, 2 * _H:3 * _H]] * _NH, axis=0)
                    vden = jnp.concatenate([v4 * mask_kv, mask_kv], axis=1)
                    cd = jnp.dot(p, vden, preferred_element_type=f32)
                    ctx_s[pl.ds(base, GR), :] = (cd[:, 0:_H]
                                                 / cd[:, _H:2 * _H])
            return carry

        jax.lax.fori_loop(0, n_grp // 8, attn_chunk, 0)

        attn = (jnp.dot(ctx_s[...], wo_ref[l], preferred_element_type=f32)
                + bo_ref[l])
        h = ln(h + attn, ln1g_ref[l], ln1b_ref[l])

        ff = (jnp.dot(h, wf1_ref[l], preferred_element_type=f32)
              + bf1_ref[l])
        ff = 0.5 * ff * (1.0 + jax.lax.erf(ff * inv_sqrt2))
        ff = (jnp.dot(ff, wf2_ref[l], preferred_element_type=f32)
              + bf2_ref[l])
        h = ln(h + ff, ln2g_ref[l], ln2b_ref[l])

    cls = h.reshape(_BB, _S, _H)[:, 0, :]
    y_ref[...] = (jnp.dot(cls, wp_ref[...], preferred_element_type=f32)
                  + bp_ref[...])


def kernel(tok_emb, pos_emb, emb_ln_g, emb_ln_b, wqkv, bqkv, wo, bo,
           ln1_g, ln1_b, wf1, bf1, wf2, bf2, ln2_g, ln2_b, wp, bp, tokens):
    B, S = tokens.shape
    grid = B // _BB

    def full(shape):
        return pl.BlockSpec(shape, lambda b: (0,) * len(shape))

    y = pl.pallas_call(
        _fwd_kernel,
        out_shape=jax.ShapeDtypeStruct((B, _NC), jnp.float32),
        grid=(grid,),
        in_specs=[
            pl.BlockSpec((_BB, S), lambda b: (b, 0)),
            full((_VOCAB, _H)), full((_S, _H)),
            full((1, _H)), full((1, _H)),
            full((_NL, _H, 3 * _H)), full((_NL, 1, 3 * _H)),
            full((_NL, _H, _H)), full((_NL, 1, _H)),
            full((_NL, 1, _H)), full((_NL, 1, _H)),
            full((_NL, _H, _FF)), full((_NL, 1, _FF)),
            full((_NL, _FF, _H)), full((_NL, 1, _H)),
            full((_NL, 1, _H)), full((_NL, 1, _H)),
            full((_H, _NC)), full((1, _NC)),
        ],
        out_specs=pl.BlockSpec((_BB, _NC), lambda b: (b, 0)),
        scratch_shapes=[
            pltpu.VMEM((_BB * _S, 3 * _H), jnp.float32),
            pltpu.VMEM((_BB * _S, _H), jnp.float32),
            pltpu.VMEM((_H, _BB * _S), jnp.float32),
        ],
        compiler_params=pltpu.CompilerParams(
            dimension_semantics=("parallel",)),
    )(tokens, tok_emb, pos_emb, emb_ln_g, emb_ln_b,
      wqkv, bqkv, wo, bo, ln1_g, ln1_b,
      wf1, bf1, wf2, bf2, ln2_g, ln2_b, wp, bp)
    return y


# speedup vs baseline: 1.2069x; 1.2069x over previous
import functools
import math

import jax
import jax.numpy as jnp
from jax.experimental import pallas as pl
from jax.experimental.pallas import tpu as pltpu

_VOCAB = 64
_S = 16
_H = 32
_NH = 4
_HD = 8
_FF = 64
_NL = 2
_NC = 3
_BB = 512
_G = 4


def _fwd_kernel(tok_ref, te_ref, pe_ref, eg_ref, eb_ref,
                wqkv_ref, bqkv_ref, wo_ref, bo_ref,
                ln1g_ref, ln1b_ref, wf1_ref, bf1_ref, wf2_ref, bf2_ref,
                ln2g_ref, ln2b_ref, wp_ref, bp_ref,
                y_ref, qkv_s, ctx_s, kt_s):
    R = _BB * _S
    scale = 1.0 / math.sqrt(_HD)
    inv_sqrt2 = 1.0 / math.sqrt(2.0)
    GR = _G * _S
    GC = _NH * GR

    f32 = jnp.float32

    def ln(x, g, b):
        mu = jnp.mean(x, axis=-1, keepdims=True)
        var = jnp.mean(jnp.square(x - mu), axis=-1, keepdims=True)
        return (x - mu) * jax.lax.rsqrt(var + 1e-12) * g + b

    r_i = jax.lax.broadcasted_iota(jnp.int32, (GC, _H), 0)
    c_i = jax.lax.broadcasted_iota(jnp.int32, (GC, _H), 1)
    mask_kv = (r_i // GR == c_i // _HD).astype(f32)
    t_r = jax.lax.broadcasted_iota(jnp.int32, (_H, GC), 0)
    t_c = jax.lax.broadcasted_iota(jnp.int32, (_H, GC), 1)
    mask_kt = jnp.where(t_r // _HD == t_c // GR, scale, 0.0).astype(f32)
    sr = jax.lax.broadcasted_iota(jnp.int32, (GR, GC), 0)
    sc = jax.lax.broadcasted_iota(jnp.int32, (GR, GC), 1)
    neg = jnp.where(sr // _S == (sc // _S) % _G, 0.0, -1e30).astype(f32)

    tok = tok_ref[...]
    voc = jax.lax.broadcasted_iota(jnp.int32, (_BB, _S, _VOCAB), 2)
    onehot = (tok[:, :, None] == voc).astype(f32).reshape(R, _VOCAB)
    emb = jnp.dot(onehot, te_ref[...], preferred_element_type=f32)
    pos = jnp.broadcast_to(pe_ref[...][None], (_BB, _S, _H)).reshape(R, _H)
    h = ln(emb + pos, eg_ref[...], eb_ref[...])

    n_grp = _BB // _G
    for l in range(_NL):
        qkv = (jnp.dot(h, wqkv_ref[l], preferred_element_type=f32)
               + bqkv_ref[l])
        qkv_s[...] = qkv
        kt_s[...] = qkv[:, _H:2 * _H].T

        def attn_chunk(c, carry):
            for uu in range(8):
                ck = c * 8 + uu
                ktc = kt_s[:, pl.ds(ck * 2 * GR, 2 * GR)]
                for u in range(2):
                    g = ck * 2 + u
                    base = g * GR
                    blk = qkv_s[pl.ds(base, GR), :]
                    q = blk[:, 0:_H]
                    ktg = ktc[:, u * GR:(u + 1) * GR]
                    kt4 = jnp.concatenate([ktg] * _NH, axis=1) * mask_kt
                    s = jnp.dot(q, kt4, preferred_element_type=f32)
                    p = jnp.exp(s + neg)
                    v4 = jnp.concatenate(
                        [blk[:, 2 * _H:3 * _H]] * _NH, axis=0)
                    vden = jnp.concatenate([v4 * mask_kv, mask_kv], axis=1)
                    cd = jnp.dot(p, vden, preferred_element_type=f32)
                    ctx_s[pl.ds(base, GR), :] = (cd[:, 0:_H]
                                                 / cd[:, _H:2 * _H])
            return carry

        jax.lax.fori_loop(0, n_grp // 16, attn_chunk, 0)

        attn = (jnp.dot(ctx_s[...], wo_ref[l], preferred_element_type=f32)
                + bo_ref[l])
        h = ln(h + attn, ln1g_ref[l], ln1b_ref[l])

        ff = (jnp.dot(h, wf1_ref[l], preferred_element_type=f32)
              + bf1_ref[l])
        ff = 0.5 * ff * (1.0 + jax.lax.erf(ff * inv_sqrt2))
        ff = (jnp.dot(ff, wf2_ref[l], preferred_element_type=f32)
              + bf2_ref[l])
        h = ln(h + ff, ln2g_ref[l], ln2b_ref[l])

    cls = h.reshape(_BB, _S, _H)[:, 0, :]
    y_ref[...] = (jnp.dot(cls, wp_ref[...], preferred_element_type=f32)
                  + bp_ref[...])


def kernel(tok_emb, pos_emb, emb_ln_g, emb_ln_b, wqkv, bqkv, wo, bo,
           ln1_g, ln1_b, wf1, bf1, wf2, bf2, ln2_g, ln2_b, wp, bp, tokens):
    B, S = tokens.shape
    grid = B // _BB

    def full(shape):
        return pl.BlockSpec(shape, lambda b: (0,) * len(shape))

    y = pl.pallas_call(
        _fwd_kernel,
        out_shape=jax.ShapeDtypeStruct((B, _NC), jnp.float32),
        grid=(grid,),
        in_specs=[
            pl.BlockSpec((_BB, S), lambda b: (b, 0)),
            full((_VOCAB, _H)), full((_S, _H)),
            full((1, _H)), full((1, _H)),
            full((_NL, _H, 3 * _H)), full((_NL, 1, 3 * _H)),
            full((_NL, _H, _H)), full((_NL, 1, _H)),
            full((_NL, 1, _H)), full((_NL, 1, _H)),
            full((_NL, _H, _FF)), full((_NL, 1, _FF)),
            full((_NL, _FF, _H)), full((_NL, 1, _H)),
            full((_NL, 1, _H)), full((_NL, 1, _H)),
            full((_H, _NC)), full((1, _NC)),
        ],
        out_specs=pl.BlockSpec((_BB, _NC), lambda b: (b, 0)),
        scratch_shapes=[
            pltpu.VMEM((_BB * _S, 3 * _H), jnp.float32),
            pltpu.VMEM((_BB * _S, _H), jnp.float32),
            pltpu.VMEM((_H, _BB * _S), jnp.float32),
        ],
        compiler_params=pltpu.CompilerParams(
            dimension_semantics=("parallel",)),
    )(tokens, tok_emb, pos_emb, emb_ln_g, emb_ln_b,
      wqkv, bqkv, wo, bo, ln1_g, ln1_b,
      wf1, bf1, wf2, bf2, ln2_g, ln2_b, wp, bp)
    return y
